# baseline (speedup 1.0000x reference)
import jax
import jax.numpy as jnp
from jax.experimental import pallas as pl
from jax.experimental.pallas import tpu as pltpu

BN = 2048


def _mm_kernel(x_ref, w_ref, o_ref):
    o_ref[...] = jax.lax.dot_general(
        x_ref[...],
        w_ref[...].astype(jnp.bfloat16),
        dimension_numbers=(((1,), (1,)), ((), ())),
        preferred_element_type=jnp.float32,
    )


def kernel(total_features, norm_weight):
    m, k = total_features.shape
    n = norm_weight.shape[0]
    total_features = total_features.astype(jnp.bfloat16)
    return pl.pallas_call(
        _mm_kernel,
        grid=(pl.cdiv(n, BN),),
        in_specs=[
            pl.BlockSpec((m, k), lambda j: (0, 0)),
            pl.BlockSpec((BN, k), lambda j: (j, 0)),
        ],
        out_specs=pl.BlockSpec((m, BN), lambda j: (0, j)),
        out_shape=jax.ShapeDtypeStruct((m, n), jnp.float32),
        compiler_params=pltpu.CompilerParams(
            dimension_semantics=("parallel",),
        ),
    )(total_features, norm_weight)


# speedup vs baseline: 1.0102x; 1.0102x over previous
import jax
import jax.numpy as jnp
from jax.experimental import pallas as pl
from jax.experimental.pallas import tpu as pltpu

BN = 4096


def _mm_kernel(x_ref, w_ref, o_ref):
    o_ref[...] = jax.lax.dot_general(
        x_ref[...],
        w_ref[...].astype(jnp.bfloat16),
        dimension_numbers=(((1,), (1,)), ((), ())),
        preferred_element_type=jnp.float32,
    )


def kernel(total_features, norm_weight):
    m, k = total_features.shape
    n = norm_weight.shape[0]
    total_features = total_features.astype(jnp.bfloat16)
    return pl.pallas_call(
        _mm_kernel,
        grid=(pl.cdiv(n, BN),),
        in_specs=[
            pl.BlockSpec((m, k), lambda j: (0, 0)),
            pl.BlockSpec((BN, k), lambda j: (j, 0)),
        ],
        out_specs=pl.BlockSpec((m, BN), lambda j: (0, j)),
        out_shape=jax.ShapeDtypeStruct((m, n), jnp.float32),
        compiler_params=pltpu.CompilerParams(
            dimension_semantics=("parallel",),
        ),
    )(total_features, norm_weight)
